# baseline (speedup 1.0000x reference)
import functools

import jax
import jax.numpy as jnp
from jax import lax
from jax.experimental import pallas as pl
from jax.experimental.pallas import tpu as pltpu
from jax.experimental.pallas import tpu_sc as plsc

BATCH = 4096
MAXLEN = 200
EMBED = 32

NC = 2
NS = 16
NW = NC * NS
T8 = MAXLEN // 8
E8 = EMBED // 8
LANES = 128
NRING = 8
NTILE = 4


def _embed_kernel(x5_hbm, tok_hbm, pos_hbm, out5_hbm,
                  idx_v, rows0_v, rows1_v, rows2_v, rows3_v,
                  rows4_v, rows5_v, rows6_v, rows7_v,
                  tile0_v, tile1_v, tile2_v, tile3_v, pos_v,
                  gsem0, gsem1, gsem2, gsem3, gsem4, gsem5, gsem6, gsem7,
                  osem0, osem1, osem2, osem3):
    wid = lax.axis_index("c") * NS + lax.axis_index("s")
    rows = (rows0_v, rows1_v, rows2_v, rows3_v,
            rows4_v, rows5_v, rows6_v, rows7_v)
    tiles = (tile0_v, tile1_v, tile2_v, tile3_v)
    gsems = (gsem0, gsem1, gsem2, gsem3, gsem4, gsem5, gsem6, gsem7)
    osems = (osem0, osem1, osem2, osem3)

    stage = [
        pltpu.async_copy(x5_hbm.at[t8, wid], idx_v.at[t8], gsem0)
        for t8 in range(T8)
    ]
    pltpu.sync_copy(pos_hbm, pos_v)
    for cp in stage:
        cp.wait()

    lane = lax.iota(jnp.int32, 16)
    ecols = (lane, lane + 16)
    e8rows = tuple(c >> 3 for c in ecols)
    esrows = tuple(c & 7 for c in ecols)

    def start_gather(t, g):
        t8 = t >> 3
        ts = t & 7
        pltpu.async_copy(tok_hbm.at[idx_v.at[t8, ts]], rows[g], gsems[g])

    def wait_gather(g):
        pltpu.make_async_copy(tok_hbm.at[idx_v.at[0, 0]], rows[g],
                              gsems[g]).wait()

    def wait_writeout(t, p):
        pltpu.make_async_copy(tiles[p], out5_hbm.at[t, :, wid],
                              osems[p]).wait()

    def transpose_add(t, g, p):
        rv = rows[g]
        tv = tiles[p]
        pvecs = (pos_v[t, pl.ds(0, 16)], pos_v[t, pl.ds(16, 16)])

        @plsc.parallel_loop(0, LANES, unroll=8)
        def diag_body(i0):
            tok = (lane + i0) & 127
            for h in range(2):
                vals = plsc.load_gather(rv, [tok, ecols[h]])
                plsc.store_scatter(tv, [e8rows[h], esrows[h], tok],
                                   vals + pvecs[h])

    for t in range(NRING - 1):
        start_gather(t, t)

    def quad_body(h, carry):
        for g in range(NRING):
            t = NRING * h + g
            p = g & 3
            wait_gather(g)

            @pl.when(t < MAXLEN - (NRING - 1))
            def _():
                start_gather(t + NRING - 1, (g + NRING - 1) % NRING)

            @pl.when(t >= NTILE)
            def _():
                wait_writeout(t - NTILE, p)

            transpose_add(t, g, p)
            pltpu.async_copy(tiles[p], out5_hbm.at[t, :, wid], osems[p])
        return carry

    lax.fori_loop(0, MAXLEN // NRING, quad_body, 0)

    for k in range(NTILE):
        t_last = MAXLEN - NTILE + k
        wait_writeout(t_last, t_last & 3)


def kernel(x, token_table, pos_table):
    x5 = x.astype(jnp.int32).T.reshape(T8, 8, NW, LANES).transpose(0, 2, 1, 3)
    mesh = plsc.VectorSubcoreMesh(core_axis_name="c", subcore_axis_name="s")
    run = functools.partial(
        pl.kernel,
        mesh=mesh,
        compiler_params=pltpu.CompilerParams(use_tc_tiling_on_sc=False,
                                             needs_layout_passes=False),
        out_type=jax.ShapeDtypeStruct((MAXLEN, E8, NW, 8, LANES),
                                      jnp.float32),
        scratch_types=(
            [pltpu.VMEM((T8, 8, LANES), jnp.int32)]
            + [pltpu.VMEM((LANES, EMBED), jnp.float32)] * NRING
            + [pltpu.VMEM((E8, 8, LANES), jnp.float32)] * NTILE
            + [pltpu.VMEM((MAXLEN, EMBED), jnp.float32)]
            + [pltpu.SemaphoreType.DMA] * (NRING + NTILE)
        ),
    )(_embed_kernel)
    out5 = run(x5, token_table, pos_table)
    return out5.transpose(2, 4, 0, 1, 3).reshape(BATCH, MAXLEN, EMBED)


# speedup vs baseline: 1.0341x; 1.0341x over previous
import functools

import jax
import jax.numpy as jnp
from jax import lax
from jax.experimental import pallas as pl
from jax.experimental.pallas import tpu as pltpu
from jax.experimental.pallas import tpu_sc as plsc

BATCH = 4096
MAXLEN = 200
EMBED = 32
VOCAB = 100000

NC = 2
NS = 16
NW = NC * NS
T8 = MAXLEN // 8
E8 = EMBED // 8
LANES = 128
NRING = 8
NTILE = 4


def _embed_kernel(x5_hbm, tok_hbm, pos_hbm, out5_hbm,
                  idx_v, rows0_v, rows1_v, rows2_v, rows3_v,
                  rows4_v, rows5_v, rows6_v, rows7_v,
                  tile0_v, tile1_v, tile2_v, tile3_v, pos_v,
                  gsem0, gsem1, gsem2, gsem3, gsem4, gsem5, gsem6, gsem7,
                  osem0, osem1, osem2, osem3):
    wid = lax.axis_index("c") * NS + lax.axis_index("s")
    rows = (rows0_v, rows1_v, rows2_v, rows3_v,
            rows4_v, rows5_v, rows6_v, rows7_v)
    tiles = (tile0_v, tile1_v, tile2_v, tile3_v)
    gsems = (gsem0, gsem1, gsem2, gsem3, gsem4, gsem5, gsem6, gsem7)
    osems = (osem0, osem1, osem2, osem3)

    stage = [
        pltpu.async_copy(x5_hbm.at[t8, wid], idx_v.at[t8], gsem0)
        for t8 in range(T8)
    ]
    pltpu.sync_copy(pos_hbm, pos_v)
    for cp in stage:
        cp.wait()

    lane = lax.iota(jnp.int32, 16)
    ecols = (lane, lane + 16)
    e8rows = tuple(c >> 3 for c in ecols)
    esrows = tuple(c & 7 for c in ecols)

    def start_gather(t, g):
        t8 = t >> 3
        ts = t & 7
        pltpu.async_copy(tok_hbm.at[idx_v.at[t8, ts]], rows[g], gsems[g])

    def wait_gather(g):
        pltpu.make_async_copy(tok_hbm.at[idx_v.at[0, 0]], rows[g],
                              gsems[g]).wait()

    def wait_writeout(t, p):
        pltpu.make_async_copy(tiles[p], out5_hbm.at[t, :, wid],
                              osems[p]).wait()

    def transpose_add(t, g, p):
        rv = rows[g]
        tv = tiles[p]
        pvecs = (pos_v[t, pl.ds(0, 16)], pos_v[t, pl.ds(16, 16)])

        @plsc.parallel_loop(0, LANES, unroll=8)
        def diag_body(i0):
            tok = (lane + i0) & 127
            for h in range(2):
                vals = plsc.load_gather(rv, [tok, ecols[h]])
                plsc.store_scatter(tv, [e8rows[h], esrows[h], tok],
                                   vals + pvecs[h])

    for t in range(NRING - 1):
        start_gather(t, t)

    def quad_body(h, carry):
        for g in range(NRING):
            t = NRING * h + g
            p = g & 3
            wait_gather(g)

            @pl.when(t < MAXLEN - (NRING - 1))
            def _():
                start_gather(t + NRING - 1, (g + NRING - 1) % NRING)

            @pl.when(t >= NTILE)
            def _():
                wait_writeout(t - NTILE, p)

            transpose_add(t, g, p)
            pltpu.async_copy(tiles[p], out5_hbm.at[t, :, wid], osems[p])
        return carry

    lax.fori_loop(0, MAXLEN // NRING, quad_body, 0)

    for k in range(NTILE):
        t_last = MAXLEN - NTILE + k
        wait_writeout(t_last, t_last & 3)


def kernel(x, token_table, pos_table):
    x5 = (x.astype(jnp.int32) * 4).T.reshape(T8, 8, NW, LANES).transpose(0, 2, 1, 3)
    tok4 = jnp.pad(token_table, ((0, 0), (0, 96))).reshape(4 * VOCAB, EMBED)
    mesh = plsc.VectorSubcoreMesh(core_axis_name="c", subcore_axis_name="s")
    run = functools.partial(
        pl.kernel,
        mesh=mesh,
        compiler_params=pltpu.CompilerParams(use_tc_tiling_on_sc=False,
                                             needs_layout_passes=False),
        out_type=jax.ShapeDtypeStruct((MAXLEN, E8, NW, 8, LANES),
                                      jnp.float32),
        scratch_types=(
            [pltpu.VMEM((T8, 8, LANES), jnp.int32)]
            + [pltpu.VMEM((LANES, EMBED), jnp.float32)] * NRING
            + [pltpu.VMEM((E8, 8, LANES), jnp.float32)] * NTILE
            + [pltpu.VMEM((MAXLEN, EMBED), jnp.float32)]
            + [pltpu.SemaphoreType.DMA] * (NRING + NTILE)
        ),
    )(_embed_kernel)
    out5 = run(x5, tok4, pos_table)
    return out5.transpose(2, 4, 0, 1, 3).reshape(BATCH, MAXLEN, EMBED)
